# Initial kernel scaffold; baseline (speedup 1.0000x reference)
import jax
import jax.numpy as jnp
from jax.experimental import pallas as pl


def kernel(input_ids, attention_mask, hidden_states, wte, wpe):
    raise NotImplementedError("write your pallas kernel here")


# speedup vs baseline: 1.0599x; 1.0599x over previous
import functools

import jax
import jax.numpy as jnp
from jax import lax
from jax.experimental import pallas as pl
from jax.experimental.pallas import tpu as pltpu
from jax.experimental.pallas import tpu_sc as plsc

_VOCAB = 100000
_MAX_POS = 2048
_D = 768
_B = 4
_S = 2048
_TOK = _B * _S
_NC = 2
_NS = 16
_NW = _NC * _NS
_PER_W = _TOK // _NW
_C = 64
_NCH = _PER_W // _C
_LANES = 16


def _emb_body(ids_hbm, wte_hbm, wpe_hbm, out_hbm, idx_v, a_v, b_v, sem):
    wid = lax.axis_index("s") * _NC + lax.axis_index("c")
    base = wid * _PER_W
    pos0 = lax.rem(base, _S)

    @pl.loop(0, _NCH)
    def _chunk(j):
        off = base + j * _C
        poff = pos0 + j * _C
        pltpu.sync_copy(ids_hbm.at[pl.ds(off, _C)], idx_v)
        gather = pltpu.async_copy(wte_hbm.at[idx_v], a_v, sem)
        pltpu.sync_copy(wpe_hbm.at[pl.ds(poff, _C)], b_v)
        gather.wait()

        @pl.loop(0, _C)
        def _row(r):
            for c in range(_D // _LANES):
                sl = pl.ds(c * _LANES, _LANES)
                plsc.addupdate(a_v.at[r, sl], b_v[r, sl])

        pltpu.sync_copy(a_v, out_hbm.at[pl.ds(off, _C)])


@functools.partial(
    pl.kernel,
    out_type=jax.ShapeDtypeStruct((_TOK, _D), jnp.float32),
    mesh=plsc.VectorSubcoreMesh(
        core_axis_name="c", subcore_axis_name="s",
        num_cores=_NC, num_subcores=_NS,
    ),
    scratch_types=[
        pltpu.VMEM((_C,), jnp.int32),
        pltpu.VMEM((_C, _D), jnp.float32),
        pltpu.VMEM((_C, _D), jnp.float32),
        pltpu.SemaphoreType.DMA,
    ],
)
def _emb_lookup(ids_hbm, wte_hbm, wpe_hbm, out_hbm, idx_v, a_v, b_v, sem):
    _emb_body(ids_hbm, wte_hbm, wpe_hbm, out_hbm, idx_v, a_v, b_v, sem)


def kernel(input_ids, attention_mask, hidden_states, wte, wpe):
    input_shape = input_ids.shape
    input_ids = input_ids.reshape(-1, input_shape[-1])
    ids_flat = input_ids.reshape(-1)
    hs = _emb_lookup(ids_flat, wte, wpe)
    hs = hs.reshape(input_ids.shape[0], input_ids.shape[1], _D)
    return (input_ids, attention_mask, hs)
